# baseline (speedup 1.0000x reference)
import jax
import jax.numpy as jnp
from jax.experimental import pallas as pl
from jax.experimental.pallas import tpu as pltpu

B = 16384
R_IN = 26
R_OUT = 11
D = 64
PIECES = (
    (0, 2, ((0, 2, 0),)),
    (6, 2, ((0, 1, 2),)),
    (12, 2, ((0, 2, 3),)),
    (14, 2, ((0, 2, 5),)),
    (16, 2, ((1, 1, 7),)),
    (20, 2, ((0, 2, 8),)),
    (22, 2, ((0, 1, 10),)),
)

W_IN = R_IN * D
W_OUT = R_OUT * D
BN = 2048


def _tc_body(*refs):
    ins, out_ref = refs[:-1], refs[-1]
    for r, (_, _, takes) in zip(ins, PIECES):
        for (off, take, dst) in takes:
            out_ref[:, pl.ds(dst * D, take * D)] = r[:, pl.ds(off * D, take * D)]


def _spec(src, w):
    return pl.BlockSpec((BN, w * D), lambda i, s=src // w: (i, s))


@jax.jit
def kernel(inputs):
    in2 = inputs.reshape(B, W_IN)
    out2 = pl.pallas_call(
        _tc_body,
        grid=(B // BN,),
        in_specs=[_spec(src, w) for (src, w, _) in PIECES],
        out_specs=pl.BlockSpec((BN, W_OUT), lambda i: (i, 0)),
        out_shape=jax.ShapeDtypeStruct((B, W_OUT), jnp.float32),
        compiler_params=pltpu.CompilerParams(
            dimension_semantics=("arbitrary",),
        ),
    )(*([in2] * len(PIECES)))
    return out2.reshape(B, R_OUT, D)
